# Initial kernel scaffold; baseline (speedup 1.0000x reference)
import jax
import jax.numpy as jnp
from jax.experimental import pallas as pl


def kernel(x, codebook, W, b):
    raise NotImplementedError("write your pallas kernel here")


# speedup vs baseline: 9.6206x; 9.6206x over previous
import functools

import jax
import jax.numpy as jnp
from jax import lax
from jax.experimental import pallas as pl
from jax.experimental.pallas import tpu as pltpu
from jax.experimental.pallas import tpu_sc as plsc

_G = 2
_TILE = 512
_CHUNK = 128


def _proj_body(x_ref, w_ref, b_ref, idx_ref, cperp_ref, pperp_ref,
               psum_acc, cnt_acc, *, num_vars, rows):
    i = pl.program_id(0)
    nsteps = pl.num_programs(0)

    @pl.when(i == 0)
    def _init():
        psum_acc[...] = jnp.zeros_like(psum_acc)
        cnt_acc[...] = jnp.zeros_like(cnt_acc)

    logits = lax.dot_general(
        x_ref[...], w_ref[...], (((1,), (1,)), ((), ())),
        preferred_element_type=jnp.float32)
    logits = logits + b_ref[...]

    iota = lax.broadcasted_iota(jnp.int32, (_TILE, num_vars), 1)
    ks = []
    for g in range(_G):
        lg = logits[:, g * num_vars:(g + 1) * num_vars]
        m = jnp.max(lg, axis=1, keepdims=True)
        k = jnp.min(jnp.where(lg == m, iota, num_vars), axis=1, keepdims=True)
        e = jnp.exp(lg - m)
        p = e * (1.0 / jnp.sum(e, axis=1, keepdims=True))
        psum_acc[g:g + 1, :] += jnp.sum(p, axis=0, keepdims=True)
        cnt_acc[g:g + 1, :] += jnp.sum(
            (iota == k).astype(jnp.float32), axis=0, keepdims=True)
        ks.append(k + g * num_vars)
    idx_ref[...] = jnp.concatenate(ks, axis=1)

    @pl.when(i == nsteps - 1)
    def _fini():
        inv_n = 1.0 / rows
        hp = cnt_acc[...] * inv_n
        ent_h = jnp.sum(hp * jnp.log(hp + 1e-7), axis=1, keepdims=True)
        cperp_ref[...] = jnp.sum(jnp.exp(-ent_h), axis=0, keepdims=True)
        ap = psum_acc[...] * inv_n
        ent_a = jnp.sum(ap * jnp.log(ap + 1e-7), axis=1, keepdims=True)
        pperp_ref[...] = jnp.sum(jnp.exp(-ent_a), axis=0, keepdims=True)


def _project_and_select(xf, W, b2, num_vars):
    rows, fsz = xf.shape
    gv = W.shape[0]
    grid = rows // _TILE
    body = functools.partial(_proj_body, num_vars=num_vars, rows=float(rows))
    return pl.pallas_call(
        body,
        grid=(grid,),
        in_specs=[
            pl.BlockSpec((_TILE, fsz), lambda i: (i, 0)),
            pl.BlockSpec((gv, fsz), lambda i: (0, 0)),
            pl.BlockSpec((1, gv), lambda i: (0, 0)),
        ],
        out_specs=[
            pl.BlockSpec((_TILE, _G), lambda i: (i, 0)),
            pl.BlockSpec((1, 1), lambda i: (0, 0)),
            pl.BlockSpec((1, 1), lambda i: (0, 0)),
        ],
        out_shape=[
            jax.ShapeDtypeStruct((rows, _G), jnp.int32),
            jax.ShapeDtypeStruct((1, 1), jnp.float32),
            jax.ShapeDtypeStruct((1, 1), jnp.float32),
        ],
        scratch_shapes=[
            pltpu.VMEM((_G, num_vars), jnp.float32),
            pltpu.VMEM((_G, num_vars), jnp.float32),
        ],
    )(xf, W, b2)


def _gather_body(table_hbm, idx_hbm, out_hbm, idx_v, rows_v, sem,
                 *, num_cores, per_w, chunks):
    wid = lax.axis_index("s") * num_cores + lax.axis_index("c")
    pltpu.sync_copy(idx_hbm.at[pl.ds(wid * chunks, chunks)], idx_v)
    cps = [
        pltpu.async_copy(table_hbm.at[idx_v.at[j]],
                         rows_v.at[pl.ds(j * _CHUNK, _CHUNK)], sem)
        for j in range(chunks)
    ]
    for c in cps:
        c.wait()
    pltpu.sync_copy(rows_v, out_hbm.at[pl.ds(wid * per_w, per_w)])


def _sc_gather(table, idx2d, n_rows):
    var_dim = table.shape[-1]
    info = plsc.get_sparse_core_info()
    nw = info.num_cores * info.num_subcores
    per_w = n_rows // nw
    chunks = per_w // _CHUNK
    mesh = plsc.VectorSubcoreMesh(core_axis_name="c", subcore_axis_name="s")
    body = functools.partial(_gather_body, num_cores=info.num_cores,
                             per_w=per_w, chunks=chunks)
    f = pl.kernel(
        body,
        out_type=jax.ShapeDtypeStruct((n_rows, var_dim), jnp.float32),
        mesh=mesh,
        scratch_types=[
            pltpu.VMEM((chunks, _CHUNK), jnp.int32),
            pltpu.VMEM((per_w, var_dim), jnp.float32),
            pltpu.SemaphoreType.DMA,
        ],
    )
    return f(table, idx2d)


def kernel(x, codebook, W, b):
    bsz, tsz, fsz = x.shape
    rows = bsz * tsz
    gv = W.shape[0]
    num_vars = gv // _G
    var_dim = codebook.shape[-1]

    xf = x.reshape(rows, fsz)
    idx, cperp, pperp = _project_and_select(xf, W, b.reshape(1, gv), num_vars)
    idx2d = idx.reshape(rows * _G // _CHUNK, _CHUNK)
    q = _sc_gather(codebook.reshape(gv, var_dim), idx2d, rows * _G)
    q = q.reshape(bsz, tsz, _G * var_dim)
    return (q, gv, cperp.reshape(()), pperp.reshape(()))


# speedup vs baseline: 17.3982x; 1.8084x over previous
import functools

import jax
import jax.numpy as jnp
from jax import lax
from jax.experimental import pallas as pl
from jax.experimental.pallas import tpu as pltpu
from jax.experimental.pallas import tpu_sc as plsc

_G = 2
_TILE = 1024
_CHUNK = 128


def _proj_body(x_ref, w_ref, b_ref, idx0_ref, idx1_ref, cperp_ref, pperp_ref,
               psum_acc, cnt_acc, *, num_vars, rows):
    i = pl.program_id(0)
    nsteps = pl.num_programs(0)

    @pl.when(i == 0)
    def _init():
        psum_acc[...] = jnp.zeros_like(psum_acc)
        cnt_acc[...] = jnp.zeros_like(cnt_acc)

    lt = lax.dot_general(
        w_ref[...], x_ref[0], (((1,), (1,)), ((), ())),
        preferred_element_type=jnp.float32)
    lt = lt + b_ref[...]

    iota = lax.broadcasted_iota(jnp.int32, (num_vars, _TILE), 0)
    ones_row = jnp.ones((1, _TILE), jnp.float32)
    for g, idx_ref in ((0, idx0_ref), (1, idx1_ref)):
        lg = lt[g * num_vars:(g + 1) * num_vars, :]
        m = jnp.max(lg, axis=0, keepdims=True)
        k = jnp.min(jnp.where(lg == m, iota, num_vars), axis=0, keepdims=True)
        e = jnp.exp(lg - m)
        w_s = 1.0 / jnp.sum(e, axis=0, keepdims=True)
        psum_acc[:, g:g + 1] += lax.dot_general(
            e, w_s, (((1,), (1,)), ((), ())),
            preferred_element_type=jnp.float32)
        oh = (iota == k).astype(jnp.float32)
        cnt_acc[:, g:g + 1] += lax.dot_general(
            oh, ones_row, (((1,), (1,)), ((), ())),
            preferred_element_type=jnp.float32)
        idx_ref[...] = jnp.reshape(k + g * num_vars, (_TILE // _CHUNK, _CHUNK))

    @pl.when(i == nsteps - 1)
    def _fini():
        inv_n = 1.0 / rows
        hp = cnt_acc[...] * inv_n
        ent_h = jnp.sum(hp * jnp.log(hp + 1e-7), axis=0, keepdims=True)
        cperp_ref[...] = jnp.sum(jnp.exp(-ent_h), axis=1, keepdims=True)
        ap = psum_acc[...] * inv_n
        ent_a = jnp.sum(ap * jnp.log(ap + 1e-7), axis=0, keepdims=True)
        pperp_ref[...] = jnp.sum(jnp.exp(-ent_a), axis=1, keepdims=True)


def _project_and_select(x, W, b2, num_vars):
    bsz, tsz, fsz = x.shape
    rows = bsz * tsz
    gv = W.shape[0]
    grid = rows // _TILE
    tiles_per_b = tsz // _TILE
    nchunks = rows // _CHUNK
    body = functools.partial(_proj_body, num_vars=num_vars, rows=float(rows))
    return pl.pallas_call(
        body,
        grid=(grid,),
        in_specs=[
            pl.BlockSpec((1, _TILE, fsz),
                         lambda i: (i // tiles_per_b, i % tiles_per_b, 0)),
            pl.BlockSpec((gv, fsz), lambda i: (0, 0)),
            pl.BlockSpec((gv, 1), lambda i: (0, 0)),
        ],
        out_specs=[
            pl.BlockSpec((_TILE // _CHUNK, _CHUNK), lambda i: (i, 0)),
            pl.BlockSpec((_TILE // _CHUNK, _CHUNK), lambda i: (i, 0)),
            pl.BlockSpec((1, 1), lambda i: (0, 0)),
            pl.BlockSpec((1, 1), lambda i: (0, 0)),
        ],
        out_shape=[
            jax.ShapeDtypeStruct((nchunks, _CHUNK), jnp.int32),
            jax.ShapeDtypeStruct((nchunks, _CHUNK), jnp.int32),
            jax.ShapeDtypeStruct((1, 1), jnp.float32),
            jax.ShapeDtypeStruct((1, 1), jnp.float32),
        ],
        scratch_shapes=[
            pltpu.VMEM((num_vars, _G), jnp.float32),
            pltpu.VMEM((num_vars, _G), jnp.float32),
        ],
    )(x, W, b2)


def _gather_body(table_hbm, idx0_hbm, idx1_hbm, out_hbm, idx_v, rows_v, sem,
                 *, num_cores, tok_w, var_dim):
    wid = lax.axis_index("s") * num_cores + lax.axis_index("c")
    cw = tok_w // _CHUNK
    pltpu.sync_copy(idx0_hbm.at[pl.ds(wid * cw, cw)], idx_v.at[pl.ds(0, cw)])
    pltpu.sync_copy(idx1_hbm.at[pl.ds(wid * cw, cw)], idx_v.at[pl.ds(cw, cw)])
    cps = [
        pltpu.async_copy(table_hbm.at[idx_v.at[g * cw + c]],
                         rows_v.at[pl.ds((g * cw + c) * _CHUNK, _CHUNK)], sem)
        for g in range(_G) for c in range(cw)
    ]
    for c in cps:
        c.wait()
    for g in range(_G):
        pltpu.sync_copy(
            rows_v.at[pl.ds(g * tok_w, tok_w)],
            out_hbm.at[pl.ds(wid * tok_w, tok_w),
                       pl.ds(g * var_dim, var_dim)])


def _sc_gather(table, idx0, idx1, n_tok):
    var_dim = table.shape[-1]
    info = plsc.get_sparse_core_info()
    nw = info.num_cores * info.num_subcores
    tok_w = n_tok // nw
    mesh = plsc.VectorSubcoreMesh(core_axis_name="c", subcore_axis_name="s")
    body = functools.partial(_gather_body, num_cores=info.num_cores,
                             tok_w=tok_w, var_dim=var_dim)
    f = pl.kernel(
        body,
        out_type=jax.ShapeDtypeStruct((n_tok, _G * var_dim), jnp.float32),
        mesh=mesh,
        scratch_types=[
            pltpu.VMEM((_G * tok_w // _CHUNK, _CHUNK), jnp.int32),
            pltpu.VMEM((_G * tok_w, var_dim), jnp.float32),
            pltpu.SemaphoreType.DMA,
        ],
    )
    return f(table, idx0, idx1)


def kernel(x, codebook, W, b):
    bsz, tsz, fsz = x.shape
    rows = bsz * tsz
    gv = W.shape[0]
    num_vars = gv // _G
    var_dim = codebook.shape[-1]

    idx0, idx1, cperp, pperp = _project_and_select(
        x, W, b.reshape(gv, 1), num_vars)
    q2d = _sc_gather(codebook.reshape(gv, var_dim), idx0, idx1, rows)
    q = q2d.reshape(bsz, tsz, _G * var_dim)
    return (q, gv, cperp.reshape(()), pperp.reshape(()))
